# Initial kernel scaffold; baseline (speedup 1.0000x reference)
import jax
import jax.numpy as jnp
from jax.experimental import pallas as pl


def kernel(featureH, featureL, batch, W_down, b_down, bn1_g, bn1_b, gcn_W, gcn_b, W_up, b_up, bn2_g, bn2_b):
    raise NotImplementedError("write your pallas kernel here")


# speedup vs baseline: 5.9959x; 5.9959x over previous
import numpy as np

import jax
import jax.numpy as jnp
from jax.experimental import pallas as pl

F32 = jnp.float32
_EPS = 1e-5

_T = 8
_NHF = 196
_NLF = 49
_NH = _T * _NHF
_NL = _T * _NLF
_NODES_F = 245
_TAIL = _T * _NODES_F - _NH


def _phase_col(f, p):
    i, j = divmod(p, 14)
    a, pi = i // 2, i % 2
    b, pj = j // 2, j % 2
    return ((pi * 2 + pj) * _T + f) * _NLF + a * 7 + b


def _build_consts():
    p_h = np.zeros((_NH, _TAIL), np.float32)
    p_l = np.zeros((_NL, _TAIL), np.float32)
    for k in range(_TAIL):
        tau, q = divmod(k, _NLF)
        a, b = divmod(q, 7)
        for pi in (0, 1):
            for pj in (0, 1):
                n = _NHF * tau + (2 * a + pi) * 14 + (2 * b + pj)
                f, pos = divmod(n, _NODES_F)
                if pos < _NHF:
                    p_h[_phase_col(f, pos), k] += 1.0
                else:
                    p_l[f * _NLF + (pos - _NHF), k] += 1.0
    s_h = np.ones((1, _NH), np.float32)
    for f in range(_T):
        for p in range(_NHF):
            if _NODES_F * f + p >= _NH:
                s_h[0, _phase_col(f, p)] = 1.0 / 9.0
    s_l = np.ones((1, _NL), np.float32)
    s_l[0, 6 * _NLF:] = 1.0 / 9.0
    m_all = np.zeros((_TAIL, 4 * 2 * _NLF), np.float32)
    for c in range(4):
        pi, pj = c // 2, c % 2
        for f in (6, 7):
            for a in range(7):
                for b in range(7):
                    p = (2 * a + pi) * 14 + (2 * b + pj)
                    n = _NODES_F * f + p
                    if n >= _NH:
                        m_all[n - _NH, c * 98 + (f - 6) * _NLF + a * 7 + b] = 1.0
    return p_h, p_l, s_h, s_l, m_all


def _down_kernel(h_ref, wd_ref, xpre_ref, sum_ref, sq_ref):
    x = jax.lax.dot_general(wd_ref[...], h_ref[0],
                            (((1,), (0,)), ((), ())),
                            preferred_element_type=F32)
    xpre_ref[0] = x

    @pl.when(pl.program_id(0) == 0)
    def _init():
        sum_ref[...] = jnp.zeros_like(sum_ref)
        sq_ref[...] = jnp.zeros_like(sq_ref)

    sum_ref[...] += jnp.sum(x, axis=1, keepdims=True)
    sq_ref[...] += jnp.sum(x * x, axis=1, keepdims=True)


def _gcn_conv_kernel(xpre_ref, l_ref, s1_ref, t1_ref, gw_ref, gb_ref,
                     wc_ref, ph_ref, pl_ref, sh_ref, sl_ref, mall_ref,
                     bz_ref, y_ref, flo_ref, sum_ref, sq_ref):
    mm = lambda a, b: jax.lax.dot_general(
        a, b, (((1,), (0,)), ((), ())), preferred_element_type=F32)
    x = jnp.maximum(xpre_ref[0] * s1_ref[...] + t1_ref[...], 0.0)
    xw_h = mm(gw_ref[...], x)
    xw_l = mm(gw_ref[...], l_ref[0])
    p_agg = mm(xw_h, ph_ref[...]) + mm(xw_l, pl_ref[...])
    base_l = xw_l * sl_ref[...] + gb_ref[...] + bz_ref[...]
    add_l = jnp.concatenate(
        [jnp.zeros_like(base_l[:, :294]),
         p_agg[:, 98:147], p_agg[:, 343:392]], axis=1)
    flo_ref[0] = base_l + (2.0 / 3.0) * add_l
    t_add = mm(p_agg, mall_ref[...])
    f_ho = xw_h * sh_ref[...] + gb_ref[...]
    lane = jax.lax.broadcasted_iota(jnp.int32, (1, _NL), 1)
    mask_a = (lane % 49) >= 7
    mask_b = (lane % 7) != 0

    chunks = []
    for c in range(4):
        ch = f_ho[:, c * _NL:(c + 1) * _NL]
        chunks.append(jnp.concatenate(
            [ch[:, :294],
             ch[:, 294:] + (2.0 / 3.0) * t_add[:, c * 98:(c + 1) * 98]],
            axis=1))
    c0, c1, c2c, c3 = chunks

    def rolled(chunk, k, mask):
        r = jnp.concatenate([chunk[:, _NL - k:], chunk[:, :_NL - k]], axis=1)
        return jnp.where(mask, r, 0.0)

    taps = [
        rolled(c3, 8, jnp.logical_and(mask_a, mask_b)),
        rolled(c2c, 7, mask_a),
        rolled(c3, 7, mask_a),
        rolled(c1, 1, mask_b),
        c0,
        c1,
        rolled(c3, 1, mask_b),
        c2c,
        c3,
    ]
    xcat = jnp.concatenate(taps, axis=0)
    y = mm(wc_ref[...], xcat)
    y_ref[0] = y

    @pl.when(pl.program_id(0) == 0)
    def _init():
        sum_ref[...] = jnp.zeros_like(sum_ref)
        sq_ref[...] = jnp.zeros_like(sq_ref)

    sum_ref[...] += jnp.sum(y, axis=1, keepdims=True)
    sq_ref[...] += jnp.sum(y * y, axis=1, keepdims=True)


def _final_kernel(y_ref, flo_ref, s2_ref, t2_ref, o_ref):
    o_ref[0] = (jnp.maximum(y_ref[0] * s2_ref[...] + t2_ref[...], 0.0)
                + flo_ref[0])


def kernel(featureH, featureL, batch, W_down, b_down, bn1_g, bn1_b,
           gcn_W, gcn_b, W_up, b_up, bn2_g, bn2_b):
    bt, c1 = featureH.shape[0], featureH.shape[1]
    c2 = featureL.shape[1]
    G = bt // _T

    h_p = (featureH.reshape(G, _T, c1, 7, 2, 7, 2)
           .transpose(0, 2, 4, 6, 1, 3, 5).reshape(G, c1, _NH))
    l_p = (featureL.reshape(G, _T, c2, _NLF)
           .transpose(0, 2, 1, 3).reshape(G, c2, _NL))
    w_cat = W_up.transpose(0, 2, 3, 1).reshape(c2, 9 * c2)

    p_h, p_l, s_h, s_l, m_all = _build_consts()
    p_h, p_l = jnp.asarray(p_h), jnp.asarray(p_l)
    s_h, s_l = jnp.asarray(s_h), jnp.asarray(s_l)
    m_all = jnp.asarray(m_all)

    xpre, sum1, sq1 = pl.pallas_call(
        _down_kernel,
        grid=(G,),
        in_specs=[
            pl.BlockSpec((1, c1, _NH), lambda i: (i, 0, 0)),
            pl.BlockSpec((c2, c1), lambda i: (0, 0)),
        ],
        out_specs=[
            pl.BlockSpec((1, c2, _NH), lambda i: (i, 0, 0)),
            pl.BlockSpec((c2, 1), lambda i: (0, 0)),
            pl.BlockSpec((c2, 1), lambda i: (0, 0)),
        ],
        out_shape=[
            jax.ShapeDtypeStruct((G, c2, _NH), F32),
            jax.ShapeDtypeStruct((c2, 1), F32),
            jax.ShapeDtypeStruct((c2, 1), F32),
        ],
    )(h_p, W_down)

    n1 = float(bt * _NHF)
    mean1 = sum1 / n1
    var1 = sq1 / n1 - mean1 * mean1
    s1 = bn1_g[:, None] * jax.lax.rsqrt(var1 + _EPS)
    t1 = bn1_b[:, None] - mean1 * s1

    bz = (jnp.asarray(batch) - 8).astype(F32).reshape(1, 1)

    y, flo, sum2, sq2 = pl.pallas_call(
        _gcn_conv_kernel,
        grid=(G,),
        in_specs=[
            pl.BlockSpec((1, c2, _NH), lambda i: (i, 0, 0)),
            pl.BlockSpec((1, c2, _NL), lambda i: (i, 0, 0)),
            pl.BlockSpec((c2, 1), lambda i: (0, 0)),
            pl.BlockSpec((c2, 1), lambda i: (0, 0)),
            pl.BlockSpec((c2, c2), lambda i: (0, 0)),
            pl.BlockSpec((c2, 1), lambda i: (0, 0)),
            pl.BlockSpec((c2, 9 * c2), lambda i: (0, 0)),
            pl.BlockSpec((_NH, _TAIL), lambda i: (0, 0)),
            pl.BlockSpec((_NL, _TAIL), lambda i: (0, 0)),
            pl.BlockSpec((1, _NH), lambda i: (0, 0)),
            pl.BlockSpec((1, _NL), lambda i: (0, 0)),
            pl.BlockSpec((_TAIL, 4 * 98), lambda i: (0, 0)),
            pl.BlockSpec((1, 1), lambda i: (0, 0)),
        ],
        out_specs=[
            pl.BlockSpec((1, c2, _NL), lambda i: (i, 0, 0)),
            pl.BlockSpec((1, c2, _NL), lambda i: (i, 0, 0)),
            pl.BlockSpec((c2, 1), lambda i: (0, 0)),
            pl.BlockSpec((c2, 1), lambda i: (0, 0)),
        ],
        out_shape=[
            jax.ShapeDtypeStruct((G, c2, _NL), F32),
            jax.ShapeDtypeStruct((G, c2, _NL), F32),
            jax.ShapeDtypeStruct((c2, 1), F32),
            jax.ShapeDtypeStruct((c2, 1), F32),
        ],
    )(xpre, l_p, s1, t1, gcn_W, gcn_b[:, None], w_cat,
      p_h, p_l, s_h, s_l, m_all, bz)

    n2 = float(bt * _NLF)
    mean2 = sum2 / n2
    var2 = sq2 / n2 - mean2 * mean2
    s2 = bn2_g[:, None] * jax.lax.rsqrt(var2 + _EPS)
    t2 = bn2_b[:, None] - mean2 * s2

    out = pl.pallas_call(
        _final_kernel,
        grid=(G,),
        in_specs=[
            pl.BlockSpec((1, c2, _NL), lambda i: (i, 0, 0)),
            pl.BlockSpec((1, c2, _NL), lambda i: (i, 0, 0)),
            pl.BlockSpec((c2, 1), lambda i: (0, 0)),
            pl.BlockSpec((c2, 1), lambda i: (0, 0)),
        ],
        out_specs=pl.BlockSpec((1, c2, _NL), lambda i: (i, 0, 0)),
        out_shape=jax.ShapeDtypeStruct((G, c2, _NL), F32),
    )(y, flo, s2, t2)

    return (out.reshape(G, c2, _T, _NLF).transpose(0, 2, 1, 3)
            .reshape(bt, c2, 7, 7))
